# baseline (speedup 1.0000x reference)
import functools

import jax
import jax.numpy as jnp
from jax import lax
from jax.experimental import pallas as pl
from jax.experimental.pallas import tpu as pltpu
from jax.experimental.pallas import tpu_sc as plsc

_V = 100000
_D = 300
_G = 100
_B = 100
_BV = 10000
_NB = _V // _BV

_SHIFT = 1 << 20
_MASK = _SHIFT - 1


def _dist_body(wgt_ref, wbt_ref, x_ref, idx_out, clue_out, counts_out,
               best_ref):
    i = pl.program_id(0)
    x = x_ref[...]
    wgt = wgt_ref[...]
    wbt = wbt_ref[...]
    pg = jnp.dot(x, wgt, preferred_element_type=jnp.float32)
    pb = jnp.dot(x, wbt, preferred_element_type=jnp.float32)
    g2 = jnp.sum(wgt * wgt, axis=0, keepdims=True)
    b2 = jnp.sum(wbt * wbt, axis=0, keepdims=True)
    tg = g2 - 2.0 * pg
    tb = b2 - 2.0 * pb
    m = jnp.min(tb, axis=1, keepdims=True)
    counts = jnp.sum((tg < m).astype(jnp.int32), axis=1, keepdims=True)
    counts_out[...] = counts[None]

    rows = i * _BV + lax.broadcasted_iota(jnp.int32, (_BV, 1), 0)
    combined = counts * _SHIFT + (_MASK - rows)
    bmax = jnp.max(combined)

    @pl.when(i == 0)
    def _():
        best_ref[0] = bmax

    @pl.when(i > 0)
    def _():
        best_ref[0] = jnp.maximum(best_ref[0], bmax)

    @pl.when(i == _NB - 1)
    def _():
        best = best_ref[0]
        clue_out[0, 0] = best // _SHIFT
        idx_out[0, 0] = _MASK - (best & _MASK)


_NC = 2
_NS = 16
_NW = _NC * _NS
_RPW = 8
_NACT = (_G + _B) // _RPW
_IPAD = 512


@functools.partial(
    pl.kernel,
    mesh=plsc.VectorSubcoreMesh(core_axis_name="c", subcore_axis_name="s"),
    out_type=jax.ShapeDtypeStruct((_G + _B, _D), jnp.float32),
    scratch_types=[
        pltpu.VMEM((16,), jnp.int32),
        pltpu.VMEM((_RPW, _D), jnp.float32),
        pltpu.SemaphoreType.DMA,
    ],
)
def _sc_gather(table_hbm, idx_hbm, out_hbm, idx_v, rows_v, sem):
    wid = lax.axis_index("s") * _NC + lax.axis_index("c")

    @pl.when(wid < _NACT)
    def _():
        base = wid * _RPW
        pltpu.sync_copy(idx_hbm.at[pl.ds(base, 16)], idx_v)
        iv = idx_v[...]
        copies = [
            pltpu.async_copy(table_hbm.at[pl.ds(iv[r], 1)],
                             rows_v.at[pl.ds(r, 1)], sem)
            for r in range(_RPW)
        ]
        for c in copies:
            c.wait()
        pltpu.sync_copy(rows_v, out_hbm.at[pl.ds(base, _RPW)])


def _distance_pass(wgt, wbt, embeddings, interpret=False):
    return pl.pallas_call(
        _dist_body,
        grid=(_NB,),
        in_specs=[
            pl.BlockSpec((_D, _G), lambda i: (0, 0)),
            pl.BlockSpec((_D, _B), lambda i: (0, 0)),
            pl.BlockSpec((_BV, _D), lambda i: (i, 0)),
        ],
        out_specs=[
            pl.BlockSpec(memory_space=pltpu.SMEM),
            pl.BlockSpec(memory_space=pltpu.SMEM),
            pl.BlockSpec((1, _BV, 1), lambda i: (i, 0, 0)),
        ],
        out_shape=[
            jax.ShapeDtypeStruct((1, 1), jnp.int32),
            jax.ShapeDtypeStruct((1, 1), jnp.int32),
            jax.ShapeDtypeStruct((_NB, _BV, 1), jnp.int32),
        ],
        scratch_shapes=[pltpu.SMEM((1,), jnp.int32)],
        interpret=interpret,
    )(wgt, wbt, embeddings)


def kernel(embeddings, good_idx, bad_idx):
    pad = jnp.zeros((_IPAD - _G - _B,), jnp.int32)
    cat_idx = jnp.concatenate(
        [good_idx.astype(jnp.int32), bad_idx.astype(jnp.int32), pad])
    w = _sc_gather(embeddings, cat_idx)
    idx, clue, counts = _distance_pass(w[:_G].T, w[_G:].T, embeddings)
    return (idx[0, 0], clue[0, 0], counts.reshape(_V))


# speedup vs baseline: 1.0087x; 1.0087x over previous
import functools

import jax
import jax.numpy as jnp
from jax import lax
from jax.experimental import pallas as pl
from jax.experimental.pallas import tpu as pltpu
from jax.experimental.pallas import tpu_sc as plsc

_V = 100000
_D = 300
_G = 100
_B = 100
_BV = 10000
_NB = _V // _BV

_SHIFT = 1 << 20
_MASK = _SHIFT - 1


def _dist_body(wgt_ref, wbt_ref, x_ref, counts_out):
    x = x_ref[...]
    wgt = wgt_ref[...]
    wbt = wbt_ref[...]
    pg = jnp.dot(x, wgt, preferred_element_type=jnp.float32)
    pb = jnp.dot(x, wbt, preferred_element_type=jnp.float32)
    g2 = jnp.sum(wgt * wgt, axis=0, keepdims=True)
    b2 = jnp.sum(wbt * wbt, axis=0, keepdims=True)
    tg = g2 - 2.0 * pg
    tb = b2 - 2.0 * pb
    m = jnp.min(tb, axis=1, keepdims=True)
    counts = jnp.sum((tg < m).astype(jnp.int32), axis=1, keepdims=True)
    counts_out[...] = counts[None]


_NC = 2
_NS = 16
_NW = _NC * _NS
_RPW = 8
_NACT = (_G + _B) // _RPW
_IPAD = 512


@functools.partial(
    pl.kernel,
    mesh=plsc.VectorSubcoreMesh(core_axis_name="c", subcore_axis_name="s"),
    out_type=jax.ShapeDtypeStruct((_G + _B, _D), jnp.float32),
    scratch_types=[
        pltpu.VMEM((16,), jnp.int32),
        pltpu.VMEM((_RPW, _D), jnp.float32),
        pltpu.SemaphoreType.DMA,
    ],
)
def _sc_gather(table_hbm, idx_hbm, out_hbm, idx_v, rows_v, sem):
    wid = lax.axis_index("s") * _NC + lax.axis_index("c")

    @pl.when(wid < _NACT)
    def _():
        base = wid * _RPW
        pltpu.sync_copy(idx_hbm.at[pl.ds(base, 16)], idx_v)
        iv = idx_v[...]
        copies = [
            pltpu.async_copy(table_hbm.at[pl.ds(iv[r], 1)],
                             rows_v.at[pl.ds(r, 1)], sem)
            for r in range(_RPW)
        ]
        for c in copies:
            c.wait()
        pltpu.sync_copy(rows_v, out_hbm.at[pl.ds(base, _RPW)])


def _distance_pass(wgt, wbt, embeddings, interpret=False):
    return pl.pallas_call(
        _dist_body,
        grid=(_NB,),
        in_specs=[
            pl.BlockSpec((_D, _G), lambda i: (0, 0)),
            pl.BlockSpec((_D, _B), lambda i: (0, 0)),
            pl.BlockSpec((_BV, _D), lambda i: (i, 0)),
        ],
        out_specs=pl.BlockSpec((1, _BV, 1), lambda i: (i, 0, 0)),
        out_shape=jax.ShapeDtypeStruct((_NB, _BV, 1), jnp.int32),
        compiler_params=pltpu.CompilerParams(
            dimension_semantics=("parallel",),
        ),
        interpret=interpret,
    )(wgt, wbt, embeddings)


_AR = 100
_AC = _V // _AR


def _argmax_body(counts_ref, idx_out, clue_out):
    counts = counts_ref[...]
    rows = (lax.broadcasted_iota(jnp.int32, (_AR, _AC), 0) * _AC
            + lax.broadcasted_iota(jnp.int32, (_AR, _AC), 1))
    combined = counts * _SHIFT + (_MASK - rows)
    best = jnp.max(combined)
    clue_out[0, 0] = best // _SHIFT
    idx_out[0, 0] = _MASK - (best & _MASK)


def _argmax_pass(counts, interpret=False):
    return pl.pallas_call(
        _argmax_body,
        out_specs=[
            pl.BlockSpec(memory_space=pltpu.SMEM),
            pl.BlockSpec(memory_space=pltpu.SMEM),
        ],
        out_shape=[
            jax.ShapeDtypeStruct((1, 1), jnp.int32),
            jax.ShapeDtypeStruct((1, 1), jnp.int32),
        ],
        interpret=interpret,
    )(counts)


def kernel(embeddings, good_idx, bad_idx):
    pad = jnp.zeros((_IPAD - _G - _B,), jnp.int32)
    cat_idx = jnp.concatenate(
        [good_idx.astype(jnp.int32), bad_idx.astype(jnp.int32), pad])
    w = _sc_gather(embeddings, cat_idx)
    counts = _distance_pass(w[:_G].T, w[_G:].T, embeddings).reshape(_V)
    idx, clue = _argmax_pass(counts.reshape(_AR, _AC))
    return (idx[0, 0], clue[0, 0], counts)
